# baseline (speedup 1.0000x reference)
import functools

import jax
import jax.numpy as jnp
from jax import lax
from jax.experimental import pallas as pl
from jax.experimental.pallas import tpu as pltpu
from jax.experimental.pallas import tpu_sc as plsc

N = 320000
D = 128
NUM_SEG = 10000

NC = 2
NS = 16
COLS = D // NC
ROWS_PER_TILE = N // NS
BLK = 400
NBLK = ROWS_PER_TILE // BLK
NBUF = 3
SEG_PER_TILE = NUM_SEG // NS
ZROWS = 125

_mesh = plsc.VectorSubcoreMesh(
    core_axis_name="c", subcore_axis_name="s", num_cores=NC, num_subcores=NS
)


@functools.partial(
    pl.kernel,
    out_type=jax.ShapeDtypeStruct((NUM_SEG, D), jnp.float32),
    mesh=_mesh,
    scratch_types=[
        [pltpu.VMEM((BLK, COLS), jnp.float32) for _ in range(NBUF)],
        [pltpu.VMEM((BLK,), jnp.int32) for _ in range(NBUF)],
        [pltpu.SemaphoreType.DMA for _ in range(NBUF)],
        [pltpu.SemaphoreType.DMA for _ in range(NBUF)],
        pltpu.VMEM((ZROWS, COLS), jnp.float32),
        pltpu.VMEM_SHARED((NUM_SEG, COLS), jnp.float32),
    ],
    compiler_params=pltpu.CompilerParams(use_tc_tiling_on_sc=False),
)
def _segment_sum_sc(x_hbm, batch_hbm, out_hbm, rows, idx, gsem, ssem, zbuf, acc_sh):
    c = lax.axis_index("c")
    s = lax.axis_index("s")
    col0 = c * COLS
    row_base = s * ROWS_PER_TILE
    seg0 = s * SEG_PER_TILE

    def gather_descs(k, b):
        r0 = row_base + k * BLK
        return [
            pltpu.make_async_copy(
                x_hbm.at[pl.ds(r0, BLK), pl.ds(col0, COLS)], rows[b], gsem[b]
            ),
            pltpu.make_async_copy(
                batch_hbm.at[pl.ds(r0, BLK)], idx[b], gsem[b]
            ),
        ]

    def start_gather(k, b):
        for d in gather_descs(k, b):
            d.start()

    def wait_gather(k, b):
        for d in gather_descs(k, b):
            d.wait()

    def start_scatter(b):
        pltpu.async_copy(rows[b], acc_sh.at[idx[b]], ssem[b], add=True)

    def drain_scatter(b):
        pltpu.make_async_copy(rows[b], acc_sh.at[idx[b]], ssem[b]).wait()

    start_gather(0, 0)
    start_gather(1, 1)

    zero = jnp.zeros((16,), jnp.float32)
    cpr = COLS // 16

    def zero_body(i, carry):
        zbuf[i // cpr, pl.ds((i % cpr) * 16, 16)] = zero
        return carry

    lax.fori_loop(0, ZROWS * cpr, zero_body, 0)
    for m in range(SEG_PER_TILE // ZROWS):
        pltpu.async_copy(zbuf, acc_sh.at[pl.ds(seg0 + m * ZROWS, ZROWS)], ssem[0])
    for m in range(SEG_PER_TILE // ZROWS):
        pltpu.make_async_copy(zbuf, acc_sh.at[pl.ds(seg0 + m * ZROWS, ZROWS)], ssem[0]).wait()
    plsc.subcore_barrier()

    def process(k, b):
        wait_gather(k, b)
        start_scatter(b)

        @pl.when(k >= 1)
        def _drain():
            drain_scatter((b + 2) % NBUF)

        @pl.when(k < NBLK - 2)
        def _next():
            start_gather(k + 2, (b + 2) % NBUF)

    def outer(t, carry):
        for b in range(NBUF):
            process(t * NBUF + b, b)
        return carry

    lax.fori_loop(0, NBLK // NBUF, outer, 0)
    for k in range((NBLK // NBUF) * NBUF, NBLK):
        process(k, k % NBUF)
    drain_scatter((NBLK - 1) % NBUF)
    plsc.subcore_barrier()

    pltpu.sync_copy(
        acc_sh.at[pl.ds(seg0, SEG_PER_TILE)],
        out_hbm.at[pl.ds(seg0, SEG_PER_TILE), pl.ds(col0, COLS)],
    )


def kernel(x, batch):
    return _segment_sum_sc(x, batch)


# speedup vs baseline: 1.0024x; 1.0024x over previous
import functools

import jax
import jax.numpy as jnp
from jax import lax
from jax.experimental import pallas as pl
from jax.experimental.pallas import tpu as pltpu
from jax.experimental.pallas import tpu_sc as plsc

N = 320000
D = 128
NUM_SEG = 10000

NC = 2
NS = 16
COLS = D // NC
ROWS_PER_TILE = N // NS
BLK = 400
NBLK = ROWS_PER_TILE // BLK
NBUF = 3
SEG_PER_TILE = NUM_SEG // NS
ZROWS = 125

_mesh = plsc.VectorSubcoreMesh(
    core_axis_name="c", subcore_axis_name="s", num_cores=NC, num_subcores=NS
)


@functools.partial(
    pl.kernel,
    out_type=jax.ShapeDtypeStruct((NUM_SEG, D), jnp.float32),
    mesh=_mesh,
    scratch_types=[
        [pltpu.VMEM((BLK, COLS), jnp.float32) for _ in range(NBUF)],
        [pltpu.VMEM((BLK,), jnp.int32) for _ in range(NBUF)],
        [pltpu.SemaphoreType.DMA for _ in range(NBUF)],
        [pltpu.SemaphoreType.DMA for _ in range(NBUF)],
        pltpu.VMEM((ZROWS, COLS), jnp.float32),
        pltpu.VMEM_SHARED((NUM_SEG, COLS), jnp.float32),
    ],
    compiler_params=pltpu.CompilerParams(use_tc_tiling_on_sc=False, skip_device_barrier=True),
)
def _segment_sum_sc(x_hbm, batch_hbm, out_hbm, rows, idx, gsem, ssem, zbuf, acc_sh):
    c = lax.axis_index("c")
    s = lax.axis_index("s")
    col0 = c * COLS
    row_base = s * ROWS_PER_TILE
    seg0 = s * SEG_PER_TILE

    def gather_descs(k, b):
        r0 = row_base + k * BLK
        return [
            pltpu.make_async_copy(
                x_hbm.at[pl.ds(r0, BLK), pl.ds(col0, COLS)], rows[b], gsem[b]
            ),
            pltpu.make_async_copy(
                batch_hbm.at[pl.ds(r0, BLK)], idx[b], gsem[b]
            ),
        ]

    def start_gather(k, b):
        for d in gather_descs(k, b):
            d.start()

    def wait_gather(k, b):
        for d in gather_descs(k, b):
            d.wait()

    def start_scatter(b):
        pltpu.async_copy(rows[b], acc_sh.at[idx[b]], ssem[b], add=True)

    def drain_scatter(b):
        pltpu.make_async_copy(rows[b], acc_sh.at[idx[b]], ssem[b]).wait()

    start_gather(0, 0)
    start_gather(1, 1)

    zero = jnp.zeros((16,), jnp.float32)
    cpr = COLS // 16

    def zero_body(i, carry):
        zbuf[i // cpr, pl.ds((i % cpr) * 16, 16)] = zero
        return carry

    lax.fori_loop(0, ZROWS * cpr, zero_body, 0)
    for m in range(SEG_PER_TILE // ZROWS):
        pltpu.async_copy(zbuf, acc_sh.at[pl.ds(seg0 + m * ZROWS, ZROWS)], ssem[0])
    for m in range(SEG_PER_TILE // ZROWS):
        pltpu.make_async_copy(zbuf, acc_sh.at[pl.ds(seg0 + m * ZROWS, ZROWS)], ssem[0]).wait()
    plsc.subcore_barrier()

    def process(k, b):
        wait_gather(k, b)
        start_scatter(b)

        @pl.when(k >= 1)
        def _drain():
            drain_scatter((b + 2) % NBUF)

        @pl.when(k < NBLK - 2)
        def _next():
            start_gather(k + 2, (b + 2) % NBUF)

    def outer(t, carry):
        for b in range(NBUF):
            process(t * NBUF + b, b)
        return carry

    lax.fori_loop(0, NBLK // NBUF, outer, 0)
    for k in range((NBLK // NBUF) * NBUF, NBLK):
        process(k, k % NBUF)
    drain_scatter((NBLK - 1) % NBUF)
    plsc.subcore_barrier()

    pltpu.sync_copy(
        acc_sh.at[pl.ds(seg0, SEG_PER_TILE)],
        out_hbm.at[pl.ds(seg0, SEG_PER_TILE), pl.ds(col0, COLS)],
    )


def kernel(x, batch):
    return _segment_sum_sc(x, batch)
